# baseline (speedup 1.0000x reference)
import functools

import jax
import jax.numpy as jnp
from jax import lax
from jax.experimental import pallas as pl
from jax.experimental.pallas import tpu as pltpu
from jax.experimental.pallas import tpu_sc as plsc

_L = 16
_T = 128


def _make_sc_kernel(B, V, D, C, NC, NS):
  NW = NC * NS
  bw = B // NW
  groups = bw // _L
  vchunks = V // _L
  P = C
  WB = (C * D + 1) * D * _L

  mesh = plsc.VectorSubcoreMesh(core_axis_name="c", subcore_axis_name="s")

  @functools.partial(
      pl.kernel,
      out_type=(
          jax.ShapeDtypeStruct((P * B,), jnp.float32),
          jax.ShapeDtypeStruct((P * B,), jnp.float32),
      ),
      mesh=mesh,
      compiler_params=pltpu.CompilerParams(needs_layout_passes=False),
      scratch_types=[
          pltpu.VMEM((bw * P,), jnp.int32),
          pltpu.VMEM((bw,), jnp.int32),
          [pltpu.VMEM((V,), jnp.float32) for _ in range(D)],
          pltpu.VMEM((WB,), jnp.float32),
          [pltpu.VMEM((V,), jnp.float32) for _ in range(C * D)],
          pltpu.VMEM((bw * P,), jnp.float32),
          pltpu.VMEM((bw * P,), jnp.float32),
          [pltpu.SemaphoreType.DMA for _ in range(8)],
      ],
  )
  def k(x_hbm, x1_hbm, aux_hbm, y_hbm, y1_hbm,
        x_v, x1_v, e_v, wb_v, t_v, y_v, y1_v, sems):
    wid = lax.axis_index("s") * NC + lax.axis_index("c")
    base = wid * bw

    cp_x = pltpu.async_copy(x_hbm.at[pl.ds(base * P, bw * P)], x_v, sems[0])
    cp_x1 = pltpu.async_copy(x1_hbm.at[pl.ds(base, bw)], x1_v, sems[1])
    cp_e = [pltpu.async_copy(aux_hbm.at[pl.ds(d * V, V)], e_v[d], sems[2 + d])
            for d in range(D)]
    cp_wb = pltpu.async_copy(aux_hbm.at[pl.ds(D * V, WB)], wb_v, sems[5])
    for c in cp_e:
      c.wait()
    cp_wb.wait()

    wsp = [[wb_v[pl.ds((o * (C * D) + kk) * _L, _L)]
            for kk in range(C * D)] for o in range(D)]
    bsp = [wb_v[pl.ds((C * D * D + o) * _L, _L)] for o in range(D)]

    @plsc.parallel_loop(0, vchunks, step=1, unroll=3)
    def build(kc):
      vs = pl.ds(kc * _L, _L)
      ed = [e_v[d][vs] for d in range(D)]
      for j in range(C):
        for o in range(D):
          t = bsp[o] if j == 0 else ed[0] * wsp[o][j * D]
          if j == 0:
            t = t + ed[0] * wsp[o][0]
          for d in range(1, D):
            t = t + ed[d] * wsp[o][j * D + d]
          t_v[j * D + o][vs] = t

    cp_x1.wait()

    @plsc.parallel_loop(0, groups, step=1, unroll=8)
    def group1(g):
      off = g * _L
      pbase = (off // _T) * (_T * P) + (off % _T)
      x1g = x1_v[pl.ds(off, _L)]
      for d in range(D):
        y1_v[pl.ds(pbase + d * _T, _L)] = plsc.load_gather(e_v[d], [x1g])

    cp_y1 = pltpu.async_copy(y1_v, y1_hbm.at[pl.ds(base * P, bw * P)], sems[7])
    cp_x.wait()

    @plsc.parallel_loop(0, groups, step=1, unroll=8)
    def group(g):
      off = g * _L
      pbase = (off // _T) * (_T * P) + (off % _T)
      xj = [x_v[pl.ds(pbase + j * _T, _L)] for j in range(C)]
      for o in range(D):
        acc0 = plsc.load_gather(t_v[o], [xj[0]])
        acc1 = plsc.load_gather(t_v[D + o], [xj[1]])
        acc2 = plsc.load_gather(t_v[2 * D + o], [xj[2]])
        acc3 = plsc.load_gather(t_v[3 * D + o], [xj[3]])
        y_v[pl.ds(pbase + o * _T, _L)] = (acc0 + acc1) + (acc2 + acc3)

    cp_y = pltpu.async_copy(y_v, y_hbm.at[pl.ds(base * P, bw * P)], sems[6])
    cp_y.wait()
    cp_y1.wait()

  return k


def kernel(x, x1, emb, W, b):
  B, C = x.shape
  V, D = emb.shape
  info = plsc.get_sparse_core_info()
  NC, NS = info.num_cores, info.num_subcores
  aux = jnp.concatenate(
      [emb.T.reshape(-1), jnp.repeat(jnp.concatenate([W.reshape(-1), b]), _L)])
  k = _make_sc_kernel(B, V, D, C, NC, NS)
  xp = x.reshape(B // _T, _T, C).transpose(0, 2, 1).reshape(-1)
  yp, y1p = k(xp, x1, aux)
  unview = lambda p: (
      p.reshape(B // _T, C, _T).transpose(0, 2, 1).reshape(B, C)[:, :D])
  return (unview(yp), unview(y1p))


# speedup vs baseline: 1.0037x; 1.0037x over previous
import functools

import jax
import jax.numpy as jnp
from jax import lax
from jax.experimental import pallas as pl
from jax.experimental.pallas import tpu as pltpu
from jax.experimental.pallas import tpu_sc as plsc

_L = 16
_T = 128


def _make_sc_kernel(B, V, D, C, NC, NS):
  NW = NC * NS
  bw = B // NW
  groups = bw // _L
  vchunks = V // _L
  P = C
  WB = (C * D + 1) * D * _L

  mesh = plsc.VectorSubcoreMesh(core_axis_name="c", subcore_axis_name="s")

  @functools.partial(
      pl.kernel,
      out_type=(
          jax.ShapeDtypeStruct((P * B,), jnp.float32),
          jax.ShapeDtypeStruct((P * B,), jnp.float32),
      ),
      mesh=mesh,
      compiler_params=pltpu.CompilerParams(needs_layout_passes=False, skip_device_barrier=True),
      scratch_types=[
          pltpu.VMEM((bw * P,), jnp.int32),
          pltpu.VMEM((bw,), jnp.int32),
          [pltpu.VMEM((V,), jnp.float32) for _ in range(D)],
          pltpu.VMEM((WB,), jnp.float32),
          [pltpu.VMEM((V,), jnp.float32) for _ in range(C * D)],
          pltpu.VMEM((bw * P,), jnp.float32),
          pltpu.VMEM((bw * P,), jnp.float32),
          [pltpu.SemaphoreType.DMA for _ in range(8)],
      ],
  )
  def k(x_hbm, x1_hbm, aux_hbm, y_hbm, y1_hbm,
        x_v, x1_v, e_v, wb_v, t_v, y_v, y1_v, sems):
    wid = lax.axis_index("s") * NC + lax.axis_index("c")
    base = wid * bw

    cp_x = pltpu.async_copy(x_hbm.at[pl.ds(base * P, bw * P)], x_v, sems[0])
    cp_x1 = pltpu.async_copy(x1_hbm.at[pl.ds(base, bw)], x1_v, sems[1])
    cp_e = [pltpu.async_copy(aux_hbm.at[pl.ds(d * V, V)], e_v[d], sems[2 + d])
            for d in range(D)]
    cp_wb = pltpu.async_copy(aux_hbm.at[pl.ds(D * V, WB)], wb_v, sems[5])
    for c in cp_e:
      c.wait()
    cp_wb.wait()

    wsp = [[wb_v[pl.ds((o * (C * D) + kk) * _L, _L)]
            for kk in range(C * D)] for o in range(D)]
    bsp = [wb_v[pl.ds((C * D * D + o) * _L, _L)] for o in range(D)]

    @plsc.parallel_loop(0, vchunks, step=1, unroll=3)
    def build(kc):
      vs = pl.ds(kc * _L, _L)
      ed = [e_v[d][vs] for d in range(D)]
      for j in range(C):
        for o in range(D):
          t = bsp[o] if j == 0 else ed[0] * wsp[o][j * D]
          if j == 0:
            t = t + ed[0] * wsp[o][0]
          for d in range(1, D):
            t = t + ed[d] * wsp[o][j * D + d]
          t_v[j * D + o][vs] = t

    cp_x1.wait()

    @plsc.parallel_loop(0, groups, step=1, unroll=8)
    def group1(g):
      off = g * _L
      pbase = (off // _T) * (_T * P) + (off % _T)
      x1g = x1_v[pl.ds(off, _L)]
      for d in range(D):
        y1_v[pl.ds(pbase + d * _T, _L)] = plsc.load_gather(e_v[d], [x1g])

    cp_y1 = pltpu.async_copy(y1_v, y1_hbm.at[pl.ds(base * P, bw * P)], sems[7])
    cp_x.wait()

    @plsc.parallel_loop(0, groups, step=1, unroll=8)
    def group(g):
      off = g * _L
      pbase = (off // _T) * (_T * P) + (off % _T)
      xj = [x_v[pl.ds(pbase + j * _T, _L)] for j in range(C)]
      for o in range(D):
        acc0 = plsc.load_gather(t_v[o], [xj[0]])
        acc1 = plsc.load_gather(t_v[D + o], [xj[1]])
        acc2 = plsc.load_gather(t_v[2 * D + o], [xj[2]])
        acc3 = plsc.load_gather(t_v[3 * D + o], [xj[3]])
        y_v[pl.ds(pbase + o * _T, _L)] = (acc0 + acc1) + (acc2 + acc3)

    cp_y = pltpu.async_copy(y_v, y_hbm.at[pl.ds(base * P, bw * P)], sems[6])
    cp_y.wait()
    cp_y1.wait()

  return k


def kernel(x, x1, emb, W, b):
  B, C = x.shape
  V, D = emb.shape
  info = plsc.get_sparse_core_info()
  NC, NS = info.num_cores, info.num_subcores
  aux = jnp.concatenate(
      [emb.T.reshape(-1), jnp.repeat(jnp.concatenate([W.reshape(-1), b]), _L)])
  k = _make_sc_kernel(B, V, D, C, NC, NS)
  xp = x.reshape(B // _T, _T, C).transpose(0, 2, 1).reshape(-1)
  yp, y1p = k(xp, x1, aux)
  unview = lambda p: (
      p.reshape(B // _T, C, _T).transpose(0, 2, 1).reshape(B, C)[:, :D])
  return (unview(yp), unview(y1p))


# speedup vs baseline: 1.0188x; 1.0151x over previous
import functools

import jax
import jax.numpy as jnp
from jax import lax
from jax.experimental import pallas as pl
from jax.experimental.pallas import tpu as pltpu
from jax.experimental.pallas import tpu_sc as plsc

_L = 16
_T = 128


def _make_sc_kernel(B, V, D, C, NC, NS):
  NW = NC * NS
  bw = B // NW
  groups = bw // _L
  vchunks = V // _L
  P = C

  mesh = plsc.VectorSubcoreMesh(core_axis_name="c", subcore_axis_name="s")

  @functools.partial(
      pl.kernel,
      out_type=(
          jax.ShapeDtypeStruct((P * B,), jnp.float32),
          jax.ShapeDtypeStruct((P * B,), jnp.float32),
      ),
      mesh=mesh,
      compiler_params=pltpu.CompilerParams(needs_layout_passes=False),
      scratch_types=[
          pltpu.VMEM((bw * P,), jnp.int32),
          pltpu.VMEM((bw,), jnp.int32),
          [pltpu.VMEM((V,), jnp.float32) for _ in range(D)],
          pltpu.VMEM(((C * D + 1) * D * _L,), jnp.float32),
          [pltpu.VMEM((V,), jnp.float32) for _ in range(C * D)],
          pltpu.VMEM((bw * P,), jnp.float32),
          pltpu.VMEM((bw * P,), jnp.float32),
          [pltpu.SemaphoreType.DMA for _ in range(8)],
      ],
  )
  def k(x_hbm, x1_hbm, embT_hbm, wb_hbm, y_hbm, y1_hbm,
        x_v, x1_v, e_v, wb_v, t_v, y_v, y1_v, sems):
    wid = lax.axis_index("s") * NC + lax.axis_index("c")
    base = wid * bw

    cp_x = pltpu.async_copy(x_hbm.at[pl.ds(base * P, bw * P)], x_v, sems[0])
    cp_x1 = pltpu.async_copy(x1_hbm.at[pl.ds(base, bw)], x1_v, sems[1])
    cp_e = [pltpu.async_copy(embT_hbm.at[pl.ds(d * V, V)], e_v[d], sems[2 + d])
            for d in range(D)]
    cp_wb = pltpu.async_copy(wb_hbm, wb_v, sems[5])
    for c in cp_e:
      c.wait()
    cp_wb.wait()

    wsp = [[wb_v[pl.ds((o * (C * D) + kk) * _L, _L)]
            for kk in range(C * D)] for o in range(D)]
    bsp = [wb_v[pl.ds((C * D * D + o) * _L, _L)] for o in range(D)]
    lane = lax.iota(jnp.int32, _L)

    @plsc.parallel_loop(0, vchunks, step=1, unroll=2)
    def build(kc):
      vs = pl.ds(kc * _L, _L)
      ed = [e_v[d][vs] for d in range(D)]
      for j in range(C):
        for o in range(D):
          t = bsp[o] if j == 0 else ed[0] * wsp[o][j * D]
          if j == 0:
            t = t + ed[0] * wsp[o][0]
          for d in range(1, D):
            t = t + ed[d] * wsp[o][j * D + d]
          t_v[j * D + o][vs] = t

    cp_x.wait()
    cp_x1.wait()

    @plsc.parallel_loop(0, groups, step=1, unroll=4)
    def group(g):
      off = g * _L
      pbase = (off // _T) * (_T * P) + (off % _T)
      x1g = x1_v[pl.ds(off, _L)]
      for d in range(D):
        y1_v[pl.ds(pbase + d * _T, _L)] = plsc.load_gather(e_v[d], [x1g])
      pvec = pbase + lane
      xj = [plsc.load_gather(x_v, [pvec + j * _T]) for j in range(C)]
      for o in range(D):
        acc0 = plsc.load_gather(t_v[o], [xj[0]])
        acc1 = plsc.load_gather(t_v[D + o], [xj[1]])
        acc2 = plsc.load_gather(t_v[2 * D + o], [xj[2]])
        acc3 = plsc.load_gather(t_v[3 * D + o], [xj[3]])
        y_v[pl.ds(pbase + o * _T, _L)] = (acc0 + acc1) + (acc2 + acc3)

    cp_y = pltpu.async_copy(y_v, y_hbm.at[pl.ds(base * P, bw * P)], sems[6])
    cp_y1 = pltpu.async_copy(y1_v, y1_hbm.at[pl.ds(base * P, bw * P)], sems[7])
    cp_y.wait()
    cp_y1.wait()

  return k


def kernel(x, x1, emb, W, b):
  B, C = x.shape
  V, D = emb.shape
  info = plsc.get_sparse_core_info()
  NC, NS = info.num_cores, info.num_subcores
  wb = jnp.repeat(jnp.concatenate([W.reshape(-1), b]), _L)
  k = _make_sc_kernel(B, V, D, C, NC, NS)
  xp = x.reshape(B // _T, _T, C).transpose(0, 2, 1).reshape(-1)
  yp, y1p = k(xp, x1, emb.T.reshape(-1), wb)
  unview = lambda p: (
      p.reshape(B // _T, C, _T).transpose(0, 2, 1).reshape(B, C)[:, :D])
  return (unview(yp), unview(y1p))
